# Initial kernel scaffold; baseline (speedup 1.0000x reference)
import jax
import jax.numpy as jnp
from jax import lax
from jax.experimental import pallas as pl
from jax.experimental.pallas import tpu as pltpu
from jax.experimental.pallas import tpu_sc as plsc

N_NODES = 10000
NP = 10240
E = 320000
CH = 128
NCHUNK = E // CH
NW = 32
ITERS = (NCHUNK + NW - 1) // NW
ROWS = NP // 16

F1 = 64
A1W = 16
F2 = 40
F2P = 48

_mesh = plsc.VectorSubcoreMesh(core_axis_name="c", subcore_axis_name="s")

_DNUMS = lax.GatherDimensionNumbers(
    offset_dims=(), collapsed_slice_dims=(0,), start_index_map=(0,))


def _dg16(x, idx):
    return lax.gather(x, idx.reshape(16, 1), _DNUMS, (1,),
                      mode=lax.GatherScatterMode.PROMISE_IN_BOUNDS)


def _tc1_body(x_ref, w1_ref, ase_ref, ade_ref, h1_ref, asp_ref, adp_ref):
    h = jnp.dot(x_ref[...], w1_ref[...], preferred_element_type=jnp.float32)
    h1_ref[...] = h
    asp_ref[...] = jnp.dot(h, ase_ref[...], preferred_element_type=jnp.float32)
    adp_ref[...] = jnp.dot(h, ade_ref[...], preferred_element_type=jnp.float32)


def _tc1(xp, W1, AsE, AdE):
    BN = 2048
    return pl.pallas_call(
        _tc1_body,
        grid=(NP // BN,),
        in_specs=[pl.BlockSpec((BN, 128), lambda i: (i, 0)),
                  pl.BlockSpec((128, F1), lambda i: (0, 0)),
                  pl.BlockSpec((F1, A1W), lambda i: (0, 0)),
                  pl.BlockSpec((F1, A1W), lambda i: (0, 0))],
        out_specs=[pl.BlockSpec((BN, F1), lambda i: (i, 0)),
                   pl.BlockSpec((BN, A1W), lambda i: (i, 0)),
                   pl.BlockSpec((BN, A1W), lambda i: (i, 0))],
        out_shape=[jax.ShapeDtypeStruct((NP, F1), jnp.float32),
                   jax.ShapeDtypeStruct((NP, A1W), jnp.float32),
                   jax.ShapeDtypeStruct((NP, A1W), jnp.float32)],
    )(xp, W1, AsE, AdE)


def _sca1_body(src_h, dst_h, asp_h, adp_h, ex1_h, d0_h, d1_h,
               idx_s, idx_d, As, Ad, Exb, zb, den_sp):
    c = lax.axis_index("c")
    s = lax.axis_index("s")
    w = c * 16 + s
    zv = jnp.zeros((16,), jnp.float32)

    def _zrow(i, cc):
        zb[i, :] = zv
        return cc
    lax.fori_loop(0, ROWS, _zrow, 0)
    pltpu.sync_copy(zb, den_sp.at[pl.ds(s * ROWS, ROWS)])
    plsc.subcore_barrier()

    def _chunk(i, cc):
        cid = w + i * NW

        @pl.when(cid < NCHUNK)
        def _():
            base = cid * CH
            pltpu.sync_copy(src_h.at[pl.ds(base, CH)], idx_s)
            pltpu.sync_copy(dst_h.at[pl.ds(base, CH)], idx_d)
            pltpu.sync_copy(asp_h.at[idx_s], As)
            pltpu.sync_copy(adp_h.at[idx_d], Ad)

            def _ev(j, c2):
                a = As[j, :] + Ad[j, :]
                a = jnp.where(a >= 0.0, a, a * jnp.float32(0.2))
                Exb[j, :] = jnp.exp(a)
                return c2
            lax.fori_loop(0, CH, _ev, 0)
            pltpu.sync_copy(Exb, ex1_h.at[pl.ds(base, CH)])
            pltpu.sync_copy(Exb, den_sp.at[idx_d], add=True)
        return cc
    lax.fori_loop(0, ITERS, _chunk, 0)
    plsc.subcore_barrier()

    pltpu.sync_copy(den_sp.at[pl.ds(s * ROWS, ROWS)], zb)

    @pl.when(c == 0)
    def _():
        pltpu.sync_copy(zb, d0_h.at[pl.ds(s * ROWS, ROWS)])

    @pl.when(c == 1)
    def _():
        pltpu.sync_copy(zb, d1_h.at[pl.ds(s * ROWS, ROWS)])


_sca1 = pl.kernel(
    _sca1_body,
    out_type=[jax.ShapeDtypeStruct((E, A1W), jnp.float32),
              jax.ShapeDtypeStruct((NP, A1W), jnp.float32),
              jax.ShapeDtypeStruct((NP, A1W), jnp.float32)],
    mesh=_mesh,
    scratch_types=[pltpu.VMEM((CH,), jnp.int32),
                   pltpu.VMEM((CH,), jnp.int32),
                   pltpu.VMEM((CH, A1W), jnp.float32),
                   pltpu.VMEM((CH, A1W), jnp.float32),
                   pltpu.VMEM((CH, A1W), jnp.float32),
                   pltpu.VMEM((ROWS, A1W), jnp.float32),
                   pltpu.VMEM_SHARED((NP, A1W), jnp.float32)],
)


def _scb1_body(src_h, dst_h, ex1_h, d0_h, d1_h, h1_h, o0_h, o1_h,
               idx_s, idx_d, Exb, G0, G1, Hb, ob, out_sp):
    c = lax.axis_index("c")
    s = lax.axis_index("s")
    w = c * 16 + s
    zv = jnp.zeros((16,), jnp.float32)

    def _zrow(i, cc):
        for q in range(4):
            ob[i, pl.ds(q * 16, 16)] = zv
        return cc
    lax.fori_loop(0, ROWS, _zrow, 0)
    pltpu.sync_copy(ob, out_sp.at[pl.ds(s * ROWS, ROWS)])
    plsc.subcore_barrier()

    lane_hi = (lax.broadcasted_iota(jnp.int32, (16,), 0) >= 8).astype(jnp.int32)
    qidx = [lane_hi + 2 * q for q in range(4)]

    def _chunk(i, cc):
        cid = w + i * NW

        @pl.when(cid < NCHUNK)
        def _():
            base = cid * CH
            pltpu.sync_copy(src_h.at[pl.ds(base, CH)], idx_s)
            pltpu.sync_copy(dst_h.at[pl.ds(base, CH)], idx_d)
            pltpu.sync_copy(ex1_h.at[pl.ds(base, CH)], Exb)
            pltpu.sync_copy(d0_h.at[idx_d], G0)
            pltpu.sync_copy(d1_h.at[idx_d], G1)
            pltpu.sync_copy(h1_h.at[idx_s], Hb)

            def _me(j, c2):
                r = Exb[j, :] / (G0[j, :] + G1[j, :] + jnp.float32(1e-16))
                for q in range(4):
                    cv = _dg16(r, qidx[q])
                    Hb[j, pl.ds(q * 16, 16)] = Hb[j, pl.ds(q * 16, 16)] * cv
                return c2
            lax.fori_loop(0, CH, _me, 0)
            pltpu.sync_copy(Hb, out_sp.at[idx_d], add=True)
        return cc
    lax.fori_loop(0, ITERS, _chunk, 0)
    plsc.subcore_barrier()

    pltpu.sync_copy(out_sp.at[pl.ds(s * ROWS, ROWS)], ob)

    @pl.when(c == 0)
    def _():
        pltpu.sync_copy(ob, o0_h.at[pl.ds(s * ROWS, ROWS)])

    @pl.when(c == 1)
    def _():
        pltpu.sync_copy(ob, o1_h.at[pl.ds(s * ROWS, ROWS)])


_scb1 = pl.kernel(
    _scb1_body,
    out_type=[jax.ShapeDtypeStruct((NP, F1), jnp.float32),
              jax.ShapeDtypeStruct((NP, F1), jnp.float32)],
    mesh=_mesh,
    scratch_types=[pltpu.VMEM((CH,), jnp.int32),
                   pltpu.VMEM((CH,), jnp.int32),
                   pltpu.VMEM((CH, A1W), jnp.float32),
                   pltpu.VMEM((CH, A1W), jnp.float32),
                   pltpu.VMEM((CH, A1W), jnp.float32),
                   pltpu.VMEM((CH, F1), jnp.float32),
                   pltpu.VMEM((ROWS, F1), jnp.float32),
                   pltpu.VMEM_SHARED((NP, F1), jnp.float32)],
)


def _tc2_body(o0_ref, o1_ref, b1_ref, w2_ref, s2_ref, d2_ref,
              h2_ref, a2s_ref, a2d_ref):
    z = o0_ref[...] + o1_ref[...] + b1_ref[...]
    act = jnp.where(z > 0.0, z, jnp.expm1(z))
    h2 = jnp.dot(act, w2_ref[...], preferred_element_type=jnp.float32)
    h2_ref[...] = h2
    a2s_ref[...] = jnp.dot(h2, s2_ref[...], preferred_element_type=jnp.float32)
    a2d_ref[...] = jnp.dot(h2, d2_ref[...], preferred_element_type=jnp.float32)


def _tc2(o0, o1, b1r, W2p, s2, d2):
    BN = 2048
    return pl.pallas_call(
        _tc2_body,
        grid=(NP // BN,),
        in_specs=[pl.BlockSpec((BN, F1), lambda i: (i, 0)),
                  pl.BlockSpec((BN, F1), lambda i: (i, 0)),
                  pl.BlockSpec((1, F1), lambda i: (0, 0)),
                  pl.BlockSpec((F1, F2P), lambda i: (0, 0)),
                  pl.BlockSpec((F2P, 1), lambda i: (0, 0)),
                  pl.BlockSpec((F2P, 1), lambda i: (0, 0))],
        out_specs=[pl.BlockSpec((BN, F2P), lambda i: (i, 0)),
                   pl.BlockSpec((BN, 1), lambda i: (i, 0)),
                   pl.BlockSpec((BN, 1), lambda i: (i, 0))],
        out_shape=[jax.ShapeDtypeStruct((NP, F2P), jnp.float32),
                   jax.ShapeDtypeStruct((NP, 1), jnp.float32),
                   jax.ShapeDtypeStruct((NP, 1), jnp.float32)],
    )(o0, o1, b1r, W2p, s2, d2)


def _sca2_body(src_h, dst_h, a2s_h, a2d_h, ex2_h, e0_h, e1_h,
               idx_s, idx_d, asb, adb, exb, zb, den_sp):
    c = lax.axis_index("c")
    s = lax.axis_index("s")
    w = c * 16 + s
    zv = jnp.zeros((16,), jnp.float32)

    pltpu.sync_copy(a2s_h, asb)
    pltpu.sync_copy(a2d_h, adb)

    def _zrow(i, cc):
        zb[pl.ds(i * 16, 16)] = zv
        return cc
    lax.fori_loop(0, ROWS // 16, _zrow, 0)
    pltpu.sync_copy(zb, den_sp.at[pl.ds(s * ROWS, ROWS)])
    plsc.subcore_barrier()

    def _chunk(i, cc):
        cid = w + i * NW

        @pl.when(cid < NCHUNK)
        def _():
            base = cid * CH
            pltpu.sync_copy(src_h.at[pl.ds(base, CH)], idx_s)
            pltpu.sync_copy(dst_h.at[pl.ds(base, CH)], idx_d)

            def _ev(j, c2):
                i16s = idx_s[pl.ds(j * 16, 16)]
                i16d = idx_d[pl.ds(j * 16, 16)]
                a = plsc.load_gather(asb, [i16s]) + plsc.load_gather(adb, [i16d])
                a = jnp.where(a >= 0.0, a, a * jnp.float32(0.2))
                exb[pl.ds(j * 16, 16)] = jnp.exp(a)
                return c2
            lax.fori_loop(0, CH // 16, _ev, 0)
            pltpu.sync_copy(exb, ex2_h.at[pl.ds(base, CH)])
            pltpu.sync_copy(exb, den_sp.at[idx_d], add=True)
        return cc
    lax.fori_loop(0, ITERS, _chunk, 0)
    plsc.subcore_barrier()

    pltpu.sync_copy(den_sp.at[pl.ds(s * ROWS, ROWS)], zb)

    @pl.when(c == 0)
    def _():
        pltpu.sync_copy(zb, e0_h.at[pl.ds(s * ROWS, ROWS)])

    @pl.when(c == 1)
    def _():
        pltpu.sync_copy(zb, e1_h.at[pl.ds(s * ROWS, ROWS)])


_sca2 = pl.kernel(
    _sca2_body,
    out_type=[jax.ShapeDtypeStruct((E,), jnp.float32),
              jax.ShapeDtypeStruct((NP,), jnp.float32),
              jax.ShapeDtypeStruct((NP,), jnp.float32)],
    mesh=_mesh,
    scratch_types=[pltpu.VMEM((CH,), jnp.int32),
                   pltpu.VMEM((CH,), jnp.int32),
                   pltpu.VMEM((NP,), jnp.float32),
                   pltpu.VMEM((NP,), jnp.float32),
                   pltpu.VMEM((CH,), jnp.float32),
                   pltpu.VMEM((ROWS,), jnp.float32),
                   pltpu.VMEM_SHARED((NP,), jnp.float32)],
)


def _scb2_body(src_h, dst_h, ex2_h, e0_h, e1_h, h2_h, o0_h, o1_h,
               idx_s, idx_d, exb, rden, tmp, Hb, ob, out_sp):
    c = lax.axis_index("c")
    s = lax.axis_index("s")
    w = c * 16 + s
    zv = jnp.zeros((16,), jnp.float32)

    pltpu.sync_copy(e0_h, rden)
    pltpu.sync_copy(e1_h, tmp)

    def _r(i, cc):
        v = rden[pl.ds(i * 16, 16)] + tmp[pl.ds(i * 16, 16)] + jnp.float32(1e-16)
        rden[pl.ds(i * 16, 16)] = 1.0 / v
        return cc
    lax.fori_loop(0, NP // 16, _r, 0)

    def _zrow(i, cc):
        for q in range(3):
            ob[i, pl.ds(q * 16, 16)] = zv
        return cc
    lax.fori_loop(0, ROWS, _zrow, 0)
    pltpu.sync_copy(ob, out_sp.at[pl.ds(s * ROWS, ROWS)])
    plsc.subcore_barrier()

    def _chunk(i, cc):
        cid = w + i * NW

        @pl.when(cid < NCHUNK)
        def _():
            base = cid * CH
            pltpu.sync_copy(src_h.at[pl.ds(base, CH)], idx_s)
            pltpu.sync_copy(dst_h.at[pl.ds(base, CH)], idx_d)
            pltpu.sync_copy(ex2_h.at[pl.ds(base, CH)], exb)
            pltpu.sync_copy(h2_h.at[idx_s], Hb)

            def _oj(j, c2):
                i16d = idx_d[pl.ds(j * 16, 16)]
                rd = plsc.load_gather(rden, [i16d])
                cvec = exb[pl.ds(j * 16, 16)] * rd
                for k in range(16):
                    e = j * 16 + k
                    sp = _dg16(cvec, jnp.full((16,), k, jnp.int32))
                    for q in range(3):
                        Hb[e, pl.ds(q * 16, 16)] = Hb[e, pl.ds(q * 16, 16)] * sp
                return c2
            lax.fori_loop(0, CH // 16, _oj, 0)
            pltpu.sync_copy(Hb, out_sp.at[idx_d], add=True)
        return cc
    lax.fori_loop(0, ITERS, _chunk, 0)
    plsc.subcore_barrier()

    pltpu.sync_copy(out_sp.at[pl.ds(s * ROWS, ROWS)], ob)

    @pl.when(c == 0)
    def _():
        pltpu.sync_copy(ob, o0_h.at[pl.ds(s * ROWS, ROWS)])

    @pl.when(c == 1)
    def _():
        pltpu.sync_copy(ob, o1_h.at[pl.ds(s * ROWS, ROWS)])


_scb2 = pl.kernel(
    _scb2_body,
    out_type=[jax.ShapeDtypeStruct((NP, F2P), jnp.float32),
              jax.ShapeDtypeStruct((NP, F2P), jnp.float32)],
    mesh=_mesh,
    scratch_types=[pltpu.VMEM((CH,), jnp.int32),
                   pltpu.VMEM((CH,), jnp.int32),
                   pltpu.VMEM((CH,), jnp.float32),
                   pltpu.VMEM((NP,), jnp.float32),
                   pltpu.VMEM((NP,), jnp.float32),
                   pltpu.VMEM((CH, F2P), jnp.float32),
                   pltpu.VMEM((ROWS, F2P), jnp.float32),
                   pltpu.VMEM_SHARED((NP, F2P), jnp.float32)],
)


def _tc3_body(q0_ref, q1_ref, b2_ref, out_ref):
    z = q0_ref[...] + q1_ref[...]
    out_ref[...] = z[:, :F2] + b2_ref[...]


def _tc3(q0, q1, b2r):
    BN = 2048
    return pl.pallas_call(
        _tc3_body,
        grid=(NP // BN,),
        in_specs=[pl.BlockSpec((BN, F2P), lambda i: (i, 0)),
                  pl.BlockSpec((BN, F2P), lambda i: (i, 0)),
                  pl.BlockSpec((1, F2), lambda i: (0, 0))],
        out_specs=pl.BlockSpec((BN, F2), lambda i: (i, 0)),
        out_shape=jax.ShapeDtypeStruct((NP, F2), jnp.float32),
    )(q0, q1, b2r)


def kernel(x, edge_index, W1, att_src1, att_dst1, b1, W2, att_src2, att_dst2, b2):
    f32 = jnp.float32
    src = edge_index[0]
    dst = edge_index[1]
    xp = jnp.pad(x, ((0, NP - N_NODES), (0, 0)))

    col = jnp.repeat(jnp.arange(8, dtype=jnp.int32), 8)
    rows = jnp.arange(F1, dtype=jnp.int32)
    AsE = jnp.zeros((F1, A1W), f32).at[rows, col].set(att_src1.reshape(F1))
    AdE = jnp.zeros((F1, A1W), f32).at[rows, col].set(att_dst1.reshape(F1))

    h1, asp, adp = _tc1(xp, W1, AsE, AdE)
    ex1, d0, d1 = _sca1(src, dst, asp, adp)
    o0, o1 = _scb1(src, dst, ex1, d0, d1, h1)

    W2p = jnp.pad(W2, ((0, 0), (0, F2P - F2)))
    s2 = jnp.pad(att_src2.reshape(F2), (0, F2P - F2)).reshape(F2P, 1)
    d2 = jnp.pad(att_dst2.reshape(F2), (0, F2P - F2)).reshape(F2P, 1)
    h2p, a2s, a2d = _tc2(o0, o1, b1.reshape(1, F1), W2p, s2, d2)

    ex2, e0, e1 = _sca2(src, dst, a2s.reshape(NP), a2d.reshape(NP))
    q0, q1 = _scb2(src, dst, ex2, e0, e1, h2p)
    out = _tc3(q0, q1, b2.reshape(1, F2))
    return out[:N_NODES]


# speedup vs baseline: 43.7967x; 43.7967x over previous
import jax
import jax.numpy as jnp
from jax import lax
from jax.experimental import pallas as pl
from jax.experimental.pallas import tpu as pltpu
from jax.experimental.pallas import tpu_sc as plsc

N_NODES = 10000
NP = 10240
E = 320000
CH = 128
NCHUNK = E // CH
NW = 32
ITERS = (NCHUNK + NW - 1) // NW
ROWS = NP // 16

F1 = 64
A1W = 16
F2 = 40
F2P = 48

_mesh = plsc.VectorSubcoreMesh(core_axis_name="c", subcore_axis_name="s",
                               num_cores=2, num_subcores=16)
_sc_params = pltpu.CompilerParams(use_tc_tiling_on_sc=False,
                                  needs_layout_passes=False)

_DNUMS = lax.GatherDimensionNumbers(
    offset_dims=(), collapsed_slice_dims=(0,), start_index_map=(0,))


def _dg16(x, idx):
    return lax.gather(x, idx.reshape(16, 1), _DNUMS, (1,),
                      mode=lax.GatherScatterMode.PROMISE_IN_BOUNDS)


def _tc1_body(x_ref, w1_ref, ase_ref, ade_ref, h1_ref, asp_ref, adp_ref):
    h = jnp.dot(x_ref[...], w1_ref[...], preferred_element_type=jnp.float32)
    h1_ref[...] = h
    asp_ref[...] = jnp.dot(h, ase_ref[...], preferred_element_type=jnp.float32)
    adp_ref[...] = jnp.dot(h, ade_ref[...], preferred_element_type=jnp.float32)


def _tc1(xp, W1, AsE, AdE):
    BN = 2048
    return pl.pallas_call(
        _tc1_body,
        grid=(NP // BN,),
        in_specs=[pl.BlockSpec((BN, 128), lambda i: (i, 0)),
                  pl.BlockSpec((128, F1), lambda i: (0, 0)),
                  pl.BlockSpec((F1, A1W), lambda i: (0, 0)),
                  pl.BlockSpec((F1, A1W), lambda i: (0, 0))],
        out_specs=[pl.BlockSpec((BN, F1), lambda i: (i, 0)),
                   pl.BlockSpec((BN, A1W), lambda i: (i, 0)),
                   pl.BlockSpec((BN, A1W), lambda i: (i, 0))],
        out_shape=[jax.ShapeDtypeStruct((NP, F1), jnp.float32),
                   jax.ShapeDtypeStruct((NP, A1W), jnp.float32),
                   jax.ShapeDtypeStruct((NP, A1W), jnp.float32)],
    )(xp, W1, AsE, AdE)


def _sca1_body(src_h, dst_h, asp_h, adp_h, ex1_h, d0_h, d1_h,
               idx_s, idx_d, As, Ad, Exb, zb, den_sp):
    c = lax.axis_index("c")
    s = lax.axis_index("s")
    w = c * 16 + s
    zv = jnp.zeros((16,), jnp.float32)

    def _zrow(i, cc):
        zb[i, :] = zv
        return cc
    lax.fori_loop(0, ROWS, _zrow, 0)
    pltpu.sync_copy(zb, den_sp.at[pl.ds(s * ROWS, ROWS)])
    plsc.subcore_barrier()

    def _chunk(i, cc):
        cid = w + i * NW

        @pl.when(cid < NCHUNK)
        def _():
            base = cid * CH
            pltpu.sync_copy(src_h.at[pl.ds(base, CH)], idx_s)
            pltpu.sync_copy(dst_h.at[pl.ds(base, CH)], idx_d)
            pltpu.sync_copy(asp_h.at[idx_s], As)
            pltpu.sync_copy(adp_h.at[idx_d], Ad)

            def _ev(j, c2):
                a = As[j, :] + Ad[j, :]
                a = jnp.where(a >= 0.0, a, a * jnp.float32(0.2))
                Exb[j, :] = jnp.exp(a)
                return c2
            lax.fori_loop(0, CH, _ev, 0)
            pltpu.sync_copy(Exb, ex1_h.at[pl.ds(base, CH)])
            pltpu.sync_copy(Exb, den_sp.at[idx_d], add=True)
        return cc
    lax.fori_loop(0, ITERS, _chunk, 0)
    plsc.subcore_barrier()

    pltpu.sync_copy(den_sp.at[pl.ds(s * ROWS, ROWS)], zb)

    @pl.when(c == 0)
    def _():
        pltpu.sync_copy(zb, d0_h.at[pl.ds(s * ROWS, ROWS)])

    @pl.when(c == 1)
    def _():
        pltpu.sync_copy(zb, d1_h.at[pl.ds(s * ROWS, ROWS)])


_sca1 = pl.kernel(
    _sca1_body,
    out_type=[jax.ShapeDtypeStruct((E, A1W), jnp.float32),
              jax.ShapeDtypeStruct((NP, A1W), jnp.float32),
              jax.ShapeDtypeStruct((NP, A1W), jnp.float32)],
    mesh=_mesh,
    compiler_params=_sc_params,
    scratch_types=[pltpu.VMEM((CH,), jnp.int32),
                   pltpu.VMEM((CH,), jnp.int32),
                   pltpu.VMEM((CH, A1W), jnp.float32),
                   pltpu.VMEM((CH, A1W), jnp.float32),
                   pltpu.VMEM((CH, A1W), jnp.float32),
                   pltpu.VMEM((ROWS, A1W), jnp.float32),
                   pltpu.VMEM_SHARED((NP, A1W), jnp.float32)],
)


def _tcr_body(d0_ref, d1_ref, r_ref):
    r_ref[...] = 1.0 / (d0_ref[...] + d1_ref[...] + jnp.float32(1e-16))


def _tcr(d0, d1, width):
    BN = 2048
    return pl.pallas_call(
        _tcr_body,
        grid=(NP // BN,),
        in_specs=[pl.BlockSpec((BN, width), lambda i: (i, 0)),
                  pl.BlockSpec((BN, width), lambda i: (i, 0))],
        out_specs=pl.BlockSpec((BN, width), lambda i: (i, 0)),
        out_shape=jax.ShapeDtypeStruct((NP, width), jnp.float32),
    )(d0, d1)


def _scb1_body(src_h, dst_h, ex1_h, r1_h, h1_h, o0_h, o1_h,
               idx_s, idx_d, Exb, Rb, Hb, ob, out_sp):
    c = lax.axis_index("c")
    s = lax.axis_index("s")
    w = c * 16 + s
    zv = jnp.zeros((16,), jnp.float32)

    def _zrow(i, cc):
        for q in range(4):
            ob[i, pl.ds(q * 16, 16)] = zv
        return cc
    lax.fori_loop(0, ROWS, _zrow, 0)
    pltpu.sync_copy(ob, out_sp.at[pl.ds(s * ROWS, ROWS)])
    plsc.subcore_barrier()

    lane_hi = lax.broadcasted_iota(jnp.int32, (16,), 0) >> 3
    qidx = [lane_hi + 2 * q for q in range(4)]

    def _chunk(i, cc):
        cid = w + i * NW

        @pl.when(cid < NCHUNK)
        def _():
            base = cid * CH
            pltpu.sync_copy(src_h.at[pl.ds(base, CH)], idx_s)
            pltpu.sync_copy(dst_h.at[pl.ds(base, CH)], idx_d)
            pltpu.sync_copy(ex1_h.at[pl.ds(base, CH)], Exb)
            pltpu.sync_copy(r1_h.at[idx_d], Rb)
            pltpu.sync_copy(h1_h.at[idx_s], Hb)

            def _me(j, c2):
                r = Exb[j, :] * Rb[j, :]
                for q in range(4):
                    cv = _dg16(r, qidx[q])
                    Hb[j, pl.ds(q * 16, 16)] = Hb[j, pl.ds(q * 16, 16)] * cv
                return c2
            lax.fori_loop(0, CH, _me, 0)
            pltpu.sync_copy(Hb, out_sp.at[idx_d], add=True)
        return cc
    lax.fori_loop(0, ITERS, _chunk, 0)
    plsc.subcore_barrier()

    pltpu.sync_copy(out_sp.at[pl.ds(s * ROWS, ROWS)], ob)

    @pl.when(c == 0)
    def _():
        pltpu.sync_copy(ob, o0_h.at[pl.ds(s * ROWS, ROWS)])

    @pl.when(c == 1)
    def _():
        pltpu.sync_copy(ob, o1_h.at[pl.ds(s * ROWS, ROWS)])


_scb1 = pl.kernel(
    _scb1_body,
    out_type=[jax.ShapeDtypeStruct((NP, F1), jnp.float32),
              jax.ShapeDtypeStruct((NP, F1), jnp.float32)],
    mesh=_mesh,
    compiler_params=_sc_params,
    scratch_types=[pltpu.VMEM((CH,), jnp.int32),
                   pltpu.VMEM((CH,), jnp.int32),
                   pltpu.VMEM((CH, A1W), jnp.float32),
                   pltpu.VMEM((CH, A1W), jnp.float32),
                   pltpu.VMEM((CH, F1), jnp.float32),
                   pltpu.VMEM((ROWS, F1), jnp.float32),
                   pltpu.VMEM_SHARED((NP, F1), jnp.float32)],
)


def _tc2_body(o0_ref, o1_ref, b1_ref, w2_ref, s2_ref, d2_ref,
              h2_ref, a2s_ref, a2d_ref):
    z = o0_ref[...] + o1_ref[...] + b1_ref[...]
    act = jnp.where(z > 0.0, z, jnp.exp(z) - 1.0)
    h2 = jnp.dot(act, w2_ref[...], preferred_element_type=jnp.float32)
    h2_ref[...] = h2
    a2s_ref[...] = jnp.dot(h2, s2_ref[...], preferred_element_type=jnp.float32)
    a2d_ref[...] = jnp.dot(h2, d2_ref[...], preferred_element_type=jnp.float32)


def _tc2(o0, o1, b1r, W2p, s2, d2):
    BN = 2048
    return pl.pallas_call(
        _tc2_body,
        grid=(NP // BN,),
        in_specs=[pl.BlockSpec((BN, F1), lambda i: (i, 0)),
                  pl.BlockSpec((BN, F1), lambda i: (i, 0)),
                  pl.BlockSpec((1, F1), lambda i: (0, 0)),
                  pl.BlockSpec((F1, F2P), lambda i: (0, 0)),
                  pl.BlockSpec((F2P, 1), lambda i: (0, 0)),
                  pl.BlockSpec((F2P, 1), lambda i: (0, 0))],
        out_specs=[pl.BlockSpec((BN, F2P), lambda i: (i, 0)),
                   pl.BlockSpec((BN, 1), lambda i: (i, 0)),
                   pl.BlockSpec((BN, 1), lambda i: (i, 0))],
        out_shape=[jax.ShapeDtypeStruct((NP, F2P), jnp.float32),
                   jax.ShapeDtypeStruct((NP, 1), jnp.float32),
                   jax.ShapeDtypeStruct((NP, 1), jnp.float32)],
    )(o0, o1, b1r, W2p, s2, d2)


def _sca2_body(src_h, dst_h, a2s_h, a2d_h, ex2_h, e0_h, e1_h,
               idx_s, idx_d, asb, adb, exb, Exw, zb, den_sp):
    c = lax.axis_index("c")
    s = lax.axis_index("s")
    w = c * 16 + s
    zv = jnp.zeros((16,), jnp.float32)
    lane0 = lax.broadcasted_iota(jnp.int32, (16,), 0) == 0

    pltpu.sync_copy(a2s_h, asb)
    pltpu.sync_copy(a2d_h, adb)

    def _zrow(i, cc):
        zb[i, :] = zv
        return cc
    lax.fori_loop(0, ROWS, _zrow, 0)
    pltpu.sync_copy(zb, den_sp.at[pl.ds(s * ROWS, ROWS)])
    plsc.subcore_barrier()

    def _chunk(i, cc):
        cid = w + i * NW

        @pl.when(cid < NCHUNK)
        def _():
            base = cid * CH
            pltpu.sync_copy(src_h.at[pl.ds(base, CH)], idx_s)
            pltpu.sync_copy(dst_h.at[pl.ds(base, CH)], idx_d)

            def _ev(j, c2):
                i16s = idx_s[pl.ds(j * 16, 16)]
                i16d = idx_d[pl.ds(j * 16, 16)]
                a = plsc.load_gather(asb, [i16s]) + plsc.load_gather(adb, [i16d])
                a = jnp.where(a >= 0.0, a, a * jnp.float32(0.2))
                ex = jnp.exp(a)
                exb[pl.ds(j * 16, 16)] = ex
                for k in range(16):
                    v = _dg16(ex, jnp.full((16,), k, jnp.int32))
                    Exw[j * 16 + k, :] = jnp.where(lane0, v, 0.0)
                return c2
            lax.fori_loop(0, CH // 16, _ev, 0)
            pltpu.sync_copy(exb, ex2_h.at[pl.ds(base, CH)])
            pltpu.sync_copy(Exw, den_sp.at[idx_d], add=True)
        return cc
    lax.fori_loop(0, ITERS, _chunk, 0)
    plsc.subcore_barrier()

    pltpu.sync_copy(den_sp.at[pl.ds(s * ROWS, ROWS)], zb)

    @pl.when(c == 0)
    def _():
        pltpu.sync_copy(zb, e0_h.at[pl.ds(s * ROWS, ROWS)])

    @pl.when(c == 1)
    def _():
        pltpu.sync_copy(zb, e1_h.at[pl.ds(s * ROWS, ROWS)])


_sca2 = pl.kernel(
    _sca2_body,
    out_type=[jax.ShapeDtypeStruct((E,), jnp.float32),
              jax.ShapeDtypeStruct((NP, A1W), jnp.float32),
              jax.ShapeDtypeStruct((NP, A1W), jnp.float32)],
    mesh=_mesh,
    compiler_params=_sc_params,
    scratch_types=[pltpu.VMEM((CH,), jnp.int32),
                   pltpu.VMEM((CH,), jnp.int32),
                   pltpu.VMEM((NP,), jnp.float32),
                   pltpu.VMEM((NP,), jnp.float32),
                   pltpu.VMEM((CH,), jnp.float32),
                   pltpu.VMEM((CH, A1W), jnp.float32),
                   pltpu.VMEM((ROWS, A1W), jnp.float32),
                   pltpu.VMEM_SHARED((NP, A1W), jnp.float32)],
)


def _scb2_body(src_h, dst_h, ex2_h, r2_h, h2_h, o0_h, o1_h,
               idx_s, idx_d, exb, rden, Hb, ob, out_sp):
    c = lax.axis_index("c")
    s = lax.axis_index("s")
    w = c * 16 + s
    zv = jnp.zeros((16,), jnp.float32)

    pltpu.sync_copy(r2_h, rden)

    def _zrow(i, cc):
        for q in range(3):
            ob[i, pl.ds(q * 16, 16)] = zv
        return cc
    lax.fori_loop(0, ROWS, _zrow, 0)
    pltpu.sync_copy(ob, out_sp.at[pl.ds(s * ROWS, ROWS)])
    plsc.subcore_barrier()

    def _chunk(i, cc):
        cid = w + i * NW

        @pl.when(cid < NCHUNK)
        def _():
            base = cid * CH
            pltpu.sync_copy(src_h.at[pl.ds(base, CH)], idx_s)
            pltpu.sync_copy(dst_h.at[pl.ds(base, CH)], idx_d)
            pltpu.sync_copy(ex2_h.at[pl.ds(base, CH)], exb)
            pltpu.sync_copy(h2_h.at[idx_s], Hb)

            def _oj(j, c2):
                i16d = idx_d[pl.ds(j * 16, 16)]
                rd = plsc.load_gather(rden, [i16d])
                cvec = exb[pl.ds(j * 16, 16)] * rd
                for k in range(16):
                    e = j * 16 + k
                    sp = _dg16(cvec, jnp.full((16,), k, jnp.int32))
                    for q in range(3):
                        Hb[e, pl.ds(q * 16, 16)] = Hb[e, pl.ds(q * 16, 16)] * sp
                return c2
            lax.fori_loop(0, CH // 16, _oj, 0)
            pltpu.sync_copy(Hb, out_sp.at[idx_d], add=True)
        return cc
    lax.fori_loop(0, ITERS, _chunk, 0)
    plsc.subcore_barrier()

    pltpu.sync_copy(out_sp.at[pl.ds(s * ROWS, ROWS)], ob)

    @pl.when(c == 0)
    def _():
        pltpu.sync_copy(ob, o0_h.at[pl.ds(s * ROWS, ROWS)])

    @pl.when(c == 1)
    def _():
        pltpu.sync_copy(ob, o1_h.at[pl.ds(s * ROWS, ROWS)])


_scb2 = pl.kernel(
    _scb2_body,
    out_type=[jax.ShapeDtypeStruct((NP, F2P), jnp.float32),
              jax.ShapeDtypeStruct((NP, F2P), jnp.float32)],
    mesh=_mesh,
    compiler_params=_sc_params,
    scratch_types=[pltpu.VMEM((CH,), jnp.int32),
                   pltpu.VMEM((CH,), jnp.int32),
                   pltpu.VMEM((CH,), jnp.float32),
                   pltpu.VMEM((NP,), jnp.float32),
                   pltpu.VMEM((CH, F2P), jnp.float32),
                   pltpu.VMEM((ROWS, F2P), jnp.float32),
                   pltpu.VMEM_SHARED((NP, F2P), jnp.float32)],
)


def _tc3_body(q0_ref, q1_ref, b2_ref, out_ref):
    z = q0_ref[...] + q1_ref[...]
    out_ref[...] = z[:, :F2] + b2_ref[...]


def _tc3(q0, q1, b2r):
    BN = 2048
    return pl.pallas_call(
        _tc3_body,
        grid=(NP // BN,),
        in_specs=[pl.BlockSpec((BN, F2P), lambda i: (i, 0)),
                  pl.BlockSpec((BN, F2P), lambda i: (i, 0)),
                  pl.BlockSpec((1, F2), lambda i: (0, 0))],
        out_specs=pl.BlockSpec((BN, F2), lambda i: (i, 0)),
        out_shape=jax.ShapeDtypeStruct((NP, F2), jnp.float32),
    )(q0, q1, b2r)


def kernel(x, edge_index, W1, att_src1, att_dst1, b1, W2, att_src2, att_dst2, b2):
    f32 = jnp.float32
    src = edge_index[0]
    dst = edge_index[1]
    xp = jnp.pad(x, ((0, NP - N_NODES), (0, 0)))

    col = jnp.repeat(jnp.arange(8, dtype=jnp.int32), 8)
    rows = jnp.arange(F1, dtype=jnp.int32)
    AsE = jnp.zeros((F1, A1W), f32).at[rows, col].set(att_src1.reshape(F1))
    AdE = jnp.zeros((F1, A1W), f32).at[rows, col].set(att_dst1.reshape(F1))

    h1, asp, adp = _tc1(xp, W1, AsE, AdE)
    ex1, d0, d1 = _sca1(src, dst, asp, adp)
    r1 = _tcr(d0, d1, A1W)
    o0, o1 = _scb1(src, dst, ex1, r1, h1)

    W2p = jnp.pad(W2, ((0, 0), (0, F2P - F2)))
    s2 = jnp.pad(att_src2.reshape(F2), (0, F2P - F2)).reshape(F2P, 1)
    d2 = jnp.pad(att_dst2.reshape(F2), (0, F2P - F2)).reshape(F2P, 1)
    h2p, a2s, a2d = _tc2(o0, o1, b1.reshape(1, F1), W2p, s2, d2)

    ex2, e0, e1 = _sca2(src, dst, a2s.reshape(NP), a2d.reshape(NP))
    r2 = _tcr(e0, e1, A1W)
    q0, q1 = _scb2(src, dst, ex2, r2[:, 0].reshape(NP), h2p)
    out = _tc3(q0, q1, b2.reshape(1, F2))
    return out[:N_NODES]
